# baseline (speedup 1.0000x reference)
import functools

import jax
import jax.numpy as jnp
from jax import lax
from jax.experimental import pallas as pl
from jax.experimental.pallas import tpu as pltpu
from jax.experimental.pallas import tpu_sc as plsc

NFIELD = 26
NDIM = 64
NROW = NFIELD * NDIM
NPAIR = NFIELD * (NFIELD - 1) // 2
LANES = 16
NC, NS = 2, 16
NW = NC * NS


def kernel(x):
    B = x.shape[0]
    assert x.shape == (B, NFIELD, NDIM)
    assert B % (NW * LANES) == 0
    groups_per_worker = B // (NW * LANES)

    mesh = plsc.VectorSubcoreMesh(
        core_axis_name="c", subcore_axis_name="s", num_cores=NC, num_subcores=NS
    )

    @functools.partial(
        pl.kernel,
        out_type=jax.ShapeDtypeStruct((B * NPAIR,), jnp.float32),
        mesh=mesh,
        scratch_types=[
            pltpu.VMEM((LANES * NROW,), jnp.float32),
            pltpu.VMEM((LANES * NPAIR,), jnp.float32),
        ],
        compiler_params=pltpu.CompilerParams(needs_layout_passes=False),
    )
    def sc_kernel(x_hbm, out_hbm, x_v, out_v):
        wid = lax.axis_index("s") * NC + lax.axis_index("c")
        biota = lax.iota(jnp.int32, LANES)
        xbase = biota * NROW
        obase = biota * NPAIR

        def group_body(g, carry):
            b0 = (wid * groups_per_worker + g) * LANES
            pltpu.sync_copy(x_hbm.at[pl.ds(b0 * NROW, LANES * NROW)], x_v)

            def i_body(i, carry_i):
                ibase = xbase + i * NDIM

                def j_body(j, carry_j):
                    jbase = xbase + j * NDIM
                    acc = jnp.zeros((LANES,), jnp.float32)
                    for d in range(NDIM):
                        va = plsc.load_gather(x_v, [ibase + d])
                        vb = plsc.load_gather(x_v, [jbase + d])
                        acc = acc + va * vb
                    p = i * (2 * NFIELD - 1 - i) // 2 + (j - i - 1)
                    plsc.store_scatter(out_v, [obase + p], acc)
                    return carry_j

                return lax.fori_loop(i + 1, NFIELD, j_body, carry_i)

            lax.fori_loop(0, NFIELD - 1, i_body, 0)
            pltpu.sync_copy(out_v, out_hbm.at[pl.ds(b0 * NPAIR, LANES * NPAIR)])
            return carry

        lax.fori_loop(0, groups_per_worker, group_body, 0)

    out_flat = sc_kernel(x.reshape(B * NROW))
    return out_flat.reshape(B, NPAIR)


# speedup vs baseline: 11.5474x; 11.5474x over previous
import functools

import jax
import jax.numpy as jnp
from jax import lax
from jax.experimental import pallas as pl
from jax.experimental.pallas import tpu as pltpu
from jax.experimental.pallas import tpu_sc as plsc

NFIELD = 26
NDIM = 64
NROW = NFIELD * NDIM
NPAIR = NFIELD * (NFIELD - 1) // 2
LANES = 16
NCHUNK = NDIM // LANES
NC, NS = 2, 16
NW = NC * NS
GROUPS = [(0, 10), (10, 20), (20, 26)]


def _pair_index(i, j):
    return i * (2 * NFIELD - 1 - i) // 2 + (j - i - 1)


def kernel(x):
    B = x.shape[0]
    assert x.shape == (B, NFIELD, NDIM)
    assert B % (NW * LANES) == 0
    groups_per_worker = B // (NW * LANES)

    mesh = plsc.VectorSubcoreMesh(
        core_axis_name="c", subcore_axis_name="s", num_cores=NC, num_subcores=NS
    )

    @functools.partial(
        pl.kernel,
        out_type=jax.ShapeDtypeStruct((B * NPAIR,), jnp.float32),
        mesh=mesh,
        scratch_types=[
            pltpu.VMEM((LANES * NROW,), jnp.float32),
            pltpu.VMEM((LANES * NPAIR + LANES,), jnp.float32),
        ],
        compiler_params=pltpu.CompilerParams(needs_layout_passes=False),
    )
    def sc_kernel(x_hbm, out_hbm, x_v, out_v):
        wid = lax.axis_index("s") * NC + lax.axis_index("c")
        lane15 = lax.iota(jnp.int32, LANES) == (LANES - 1)

        def group_body(g, carry):
            b0 = (wid * groups_per_worker + g) * LANES
            pltpu.sync_copy(x_hbm.at[pl.ds(b0 * NROW, LANES * NROW)], x_v)

            def batch_body(b, carry_b):
                xb = b * NROW
                ob = b * NPAIR

                def load_field(f):
                    return [
                        x_v[pl.ds(xb + f * NDIM + c * LANES, LANES)]
                        for c in range(NCHUNK)
                    ]

                def do_pair(vi, vj, i, j):
                    t = vi[0] * vj[0]
                    for c in range(1, NCHUNK):
                        t = t + vi[c] * vj[c]
                    s = plsc.cumsum(t)
                    plsc.store_compressed(
                        out_v.at[pl.ds(ob + _pair_index(i, j), LANES)],
                        s,
                        mask=lane15,
                    )

                for gstart, gend in GROUPS:
                    cache = {f: load_field(f) for f in range(gstart, gend)}
                    for i in range(gstart, gend):
                        for j in range(i + 1, gend):
                            do_pair(cache[i], cache[j], i, j)
                    for j in range(gend, NFIELD):
                        vj = load_field(j)
                        for i in range(gstart, gend):
                            do_pair(cache[i], vj, i, j)
                return carry_b

            lax.fori_loop(0, LANES, batch_body, 0)
            pltpu.sync_copy(
                out_v.at[pl.ds(0, LANES * NPAIR)],
                out_hbm.at[pl.ds(b0 * NPAIR, LANES * NPAIR)],
            )
            return carry

        lax.fori_loop(0, groups_per_worker, group_body, 0)

    out_flat = sc_kernel(x.reshape(B * NROW))
    return out_flat.reshape(B, NPAIR)
